# baseline (speedup 1.0000x reference)
import functools

import jax
import jax.numpy as jnp
from jax import lax
from jax.experimental import pallas as pl
from jax.experimental.pallas import tpu as pltpu
from jax.experimental.pallas import tpu_sc as plsc

NC = 2
NS = 16
L = 16
K = 128


def _round_up(x, m):
    return (x + m - 1) // m * m


_ZCHUNK = 1024


def _deg_body(src_hbm, dst_hbm, out_hbm, idx_v, deg_v, *, NCH, NPAD):
    c = lax.axis_index("c")
    s = lax.axis_index("s")
    ones = jnp.ones((L,), jnp.float32)
    zeros = jnp.zeros((L,), jnp.float32)
    for a in range(2):
        ei = src_hbm if a == 0 else dst_hbm
        pltpu.sync_copy(ei.at[c, s], idx_v)

        def zbody(i, carry):
            for u in range(8):
                deg_v[pl.ds(pl.multiple_of(i * 8 * L + u * L, L), L)] = zeros
            return carry

        lax.fori_loop(0, NPAD // (8 * L), zbody, None)

        def abody(j, carry):
            for k in range(K // L):
                idx = idx_v[j, pl.ds(k * L, L)]
                plsc.addupdate_scatter(deg_v, [idx], ones)
            return carry

        lax.fori_loop(0, NCH, abody, None)
        pltpu.sync_copy(deg_v, out_hbm.at[c, s, a])


def _make_deg_kernel(NCH, NPAD):
    mesh = plsc.VectorSubcoreMesh(core_axis_name="c", subcore_axis_name="s")
    return pl.kernel(
        functools.partial(_deg_body, NCH=NCH, NPAD=NPAD),
        out_type=jax.ShapeDtypeStruct((NC, NS, 2, NPAD), jnp.float32),
        mesh=mesh,
        compiler_params=pltpu.CompilerParams(needs_layout_passes=False),
        scratch_types=[
            pltpu.VMEM((NCH, K), jnp.int32),
            pltpu.VMEM((NPAD,), jnp.float32),
        ],
    )


def _edge_body(xs_hbm, src_hbm, dst_hbm, out_hbm, src_v, dst_v, rows_v,
               agg_sh, *, NCH0, NCH1, NPAD, D):
    RPT = NPAD // NS
    ZR = 64
    c = lax.axis_index("c")
    s = lax.axis_index("s")
    pltpu.sync_copy(src_hbm.at[c, s], src_v)
    pltpu.sync_copy(dst_hbm.at[c, s], dst_v)

    zeros = jnp.zeros((L,), jnp.float32)

    def zbody(r, carry):
        for k in range(D // L):
            rows_v[r, pl.ds(k * L, L)] = zeros
        return carry

    lax.fori_loop(0, ZR, zbody, None)
    base = s * RPT
    for t in range(RPT // ZR):
        pltpu.sync_copy(rows_v.at[pl.ds(0, ZR)],
                        agg_sh.at[pl.ds(base + t * ZR, ZR)])
    plsc.subcore_barrier()

    def chunk(j, carry):
        pltpu.sync_copy(xs_hbm.at[src_v.at[j]], rows_v)
        pltpu.sync_copy(rows_v, agg_sh.at[dst_v.at[j]], add=True)
        return carry

    nch = jnp.where(c == 0, NCH0, NCH1)
    lax.fori_loop(0, nch, chunk, None)
    plsc.subcore_barrier()
    for t in range(RPT // ZR):
        sl = pl.ds(base + t * ZR, ZR)
        pltpu.sync_copy(agg_sh.at[sl], rows_v.at[pl.ds(0, ZR)])
        pltpu.sync_copy(rows_v.at[pl.ds(0, ZR)], out_hbm.at[c, sl])


def _make_edge_kernel(NCH0, NCH1, NPAD, D):
    NCHM = max(NCH0, NCH1)
    mesh = plsc.VectorSubcoreMesh(core_axis_name="c", subcore_axis_name="s")
    return pl.kernel(
        functools.partial(_edge_body, NCH0=NCH0, NCH1=NCH1, NPAD=NPAD, D=D),
        out_type=jax.ShapeDtypeStruct((NC, NPAD, D), jnp.float32),
        mesh=mesh,
        compiler_params=pltpu.CompilerParams(needs_layout_passes=False),
        scratch_types=[
            pltpu.VMEM((NCHM, K), jnp.int32),
            pltpu.VMEM((NCHM, K), jnp.int32),
            pltpu.VMEM((K, D), jnp.float32),
            pltpu.VMEM_SHARED((NPAD, D), jnp.float32),
        ],
    )


def _stage1_body(hp_ref, gam_ref, bet_ref, w1_ref, degp_ref, xs_ref, mu_ref,
                 *, N):
    NPAD = hp_ref.shape[0]
    hp = hp_ref[...]
    deg = jnp.sum(degp_ref[...], axis=0)
    norm_out = lax.rsqrt(jnp.maximum(deg[0], 1.0))
    rid = lax.broadcasted_iota(jnp.int32, (NPAD, 1), 0)
    msk = rid < N
    mu = jnp.sum(hp, axis=0, keepdims=True) / N
    xc = jnp.where(msk, hp - mu, 0.0)
    var = jnp.sum(xc * xc, axis=0, keepdims=True) / N
    inv = lax.rsqrt(var + 1e-5)
    hb = xc * inv * gam_ref[...] + bet_ref[...]
    xw = jnp.dot(hb, w1_ref[...], preferred_element_type=jnp.float32)
    xs_ref[...] = xw * norm_out[:, None]
    mu_ref[...] = mu


def _stage2_body(p_ref, hp_ref, gam_ref, bet_ref, w2_ref, b1_ref, degp_ref,
                 h1_ref, xs2_ref, *, N):
    NPAD = hp_ref.shape[0]
    deg = jnp.sum(degp_ref[...], axis=0)
    norm_out = lax.rsqrt(jnp.maximum(deg[0], 1.0))
    norm_in = lax.rsqrt(jnp.maximum(deg[1], 1.0))
    rid = lax.broadcasted_iota(jnp.int32, (NPAD, 1), 0)
    msk = rid < N
    agg = (p_ref[0] + p_ref[1]) * norm_in[:, None] + b1_ref[...]
    h1 = jnp.where(msk, hp_ref[...] + jnp.maximum(agg, 0.0), 0.0)
    mu = jnp.sum(h1, axis=0, keepdims=True) / N
    xc = jnp.where(msk, h1 - mu, 0.0)
    var = jnp.sum(xc * xc, axis=0, keepdims=True) / N
    inv = lax.rsqrt(var + 1e-5)
    hb = xc * inv * gam_ref[...] + bet_ref[...]
    xw = jnp.dot(hb, w2_ref[...], preferred_element_type=jnp.float32)
    h1_ref[...] = h1
    xs2_ref[...] = xw * norm_out[:, None]


def _stage3_body(p_ref, h1_ref, degp_ref, b2_ref, gw_ref, gb_ref, hg_ref,
                 *, N):
    NPAD = h1_ref.shape[0]
    deg = jnp.sum(degp_ref[...], axis=0)
    norm_in = lax.rsqrt(jnp.maximum(deg[1], 1.0))
    rid = lax.broadcasted_iota(jnp.int32, (NPAD, 1), 0)
    msk = rid < N
    agg = (p_ref[0] + p_ref[1]) * norm_in[:, None] + b2_ref[...]
    h2 = jnp.where(msk, h1_ref[...] + jnp.maximum(agg, 0.0), 0.0)
    gate = jnp.sum(h2 * gw_ref[...], axis=1, keepdims=True) + gb_ref[0, 0]
    gate = jnp.where(msk, gate, -1e30)
    m = jnp.max(gate)
    e = jnp.exp(gate - m)
    hg_ref[...] = jnp.sum(e * h2, axis=0, keepdims=True) / jnp.sum(e)


def kernel(h, edge_index, gamma, beta, W1, b1, W2, b2, gate_W, gate_b):
    N, D = h.shape
    E = edge_index.shape[1]
    NW = NC * NS
    NCH = -(-E // (NW * K))
    EP = NW * NCH * K
    NPAD = _round_up(N + 1, NS * 64)

    fill = jnp.full((EP - E,), N, jnp.int32)
    src_rs = jnp.concatenate([edge_index[0], fill]).reshape(NC, NS, NCH, K)
    dst_rs = jnp.concatenate([edge_index[1], fill]).reshape(NC, NS, NCH, K)

    TOT = -(-E // (NS * K))
    NCH0 = int(TOT * 0.52)
    NCH1 = TOT - NCH0
    NCHM = max(NCH0, NCH1)
    cap0 = NS * NCH0 * K
    cap1 = NS * NCH1 * K
    fill_a = jnp.full((cap0 + cap1 - E,), N, jnp.int32)

    def asym(e):
        ep = jnp.concatenate([e, fill_a])
        a0 = ep[:cap0].reshape(NS, NCH0, K)
        a0 = jnp.concatenate(
            [a0, jnp.full((NS, NCHM - NCH0, K), N, jnp.int32)], axis=1)
        a1 = ep[cap0:].reshape(NS, NCH1, K)
        a1 = jnp.concatenate(
            [a1, jnp.full((NS, NCHM - NCH1, K), N, jnp.int32)], axis=1)
        return jnp.stack([a0, a1])

    src_as = asym(edge_index[0])
    dst_as = asym(edge_index[1])

    hp = jnp.zeros((NPAD, D), jnp.float32).at[:N, :].set(h)
    gam = gamma.reshape(1, D)
    bet = beta.reshape(1, D)
    b1r = b1.reshape(1, D)
    b2r = b2.reshape(1, D)
    gw = gate_W.reshape(1, D)
    gb = gate_b.reshape(1, 1)

    degp = _make_deg_kernel(NCH, NPAD)(src_rs, dst_rs)
    degp = degp.reshape(NW, 2, NPAD)

    xs1, mu = pl.pallas_call(
        functools.partial(_stage1_body, N=N),
        out_shape=[jax.ShapeDtypeStruct((NPAD, D), jnp.float32),
                   jax.ShapeDtypeStruct((1, D), jnp.float32)],
    )(hp, gam, bet, W1, degp)

    edge_k = _make_edge_kernel(NCH0, NCH1, NPAD, D)
    p1 = edge_k(xs1, src_as, dst_as)

    h1, xs2 = pl.pallas_call(
        functools.partial(_stage2_body, N=N),
        out_shape=[jax.ShapeDtypeStruct((NPAD, D), jnp.float32),
                   jax.ShapeDtypeStruct((NPAD, D), jnp.float32)],
    )(p1, hp, gam, bet, W2, b1r, degp)

    p2 = edge_k(xs2, src_as, dst_as)

    hg = pl.pallas_call(
        functools.partial(_stage3_body, N=N),
        out_shape=jax.ShapeDtypeStruct((1, D), jnp.float32),
    )(p2, h1, degp, b2r, gw, gb)

    return (hg, mu)


# speedup vs baseline: 1.0396x; 1.0396x over previous
import functools

import jax
import jax.numpy as jnp
from jax import lax
from jax.experimental import pallas as pl
from jax.experimental.pallas import tpu as pltpu
from jax.experimental.pallas import tpu_sc as plsc

NC = 2
NS = 16
L = 16
K = 128


def _round_up(x, m):
    return (x + m - 1) // m * m


_ZCHUNK = 1024


def _deg_body(src_hbm, dst_hbm, out_hbm, idx_v, deg_v, *, NCH, NPAD):
    c = lax.axis_index("c")
    s = lax.axis_index("s")
    ones = jnp.ones((L,), jnp.float32)
    zeros = jnp.zeros((L,), jnp.float32)
    for a in range(2):
        ei = src_hbm if a == 0 else dst_hbm
        pltpu.sync_copy(ei.at[c, s], idx_v)

        def zbody(i, carry):
            for u in range(8):
                deg_v[pl.ds(pl.multiple_of(i * 8 * L + u * L, L), L)] = zeros
            return carry

        lax.fori_loop(0, NPAD // (8 * L), zbody, None)

        def abody(j, carry):
            for k in range(K // L):
                idx = idx_v[j, pl.ds(k * L, L)]
                plsc.addupdate_scatter(deg_v, [idx], ones)
            return carry

        lax.fori_loop(0, NCH, abody, None)
        pltpu.sync_copy(deg_v, out_hbm.at[c, s, a])


def _make_deg_kernel(NCH, NPAD):
    mesh = plsc.VectorSubcoreMesh(core_axis_name="c", subcore_axis_name="s")
    return pl.kernel(
        functools.partial(_deg_body, NCH=NCH, NPAD=NPAD),
        out_type=jax.ShapeDtypeStruct((NC, NS, 2, NPAD), jnp.float32),
        mesh=mesh,
        compiler_params=pltpu.CompilerParams(needs_layout_passes=False),
        scratch_types=[
            pltpu.VMEM((NCH, K), jnp.int32),
            pltpu.VMEM((NPAD,), jnp.float32),
        ],
    )


def _edge_body(xs_hbm, src_hbm, dst_hbm, out_hbm, src_v, dst_v, rows_v,
               agg_sh, *, NCH0, NCH1, NPAD, D):
    RPT = NPAD // NS
    ZR = 64
    c = lax.axis_index("c")
    s = lax.axis_index("s")
    pltpu.sync_copy(src_hbm.at[c, s], src_v)
    pltpu.sync_copy(dst_hbm.at[c, s], dst_v)

    zeros = jnp.zeros((L,), jnp.float32)

    def zbody(r, carry):
        for k in range(D // L):
            rows_v[r, pl.ds(k * L, L)] = zeros
        return carry

    lax.fori_loop(0, ZR, zbody, None)
    base = s * RPT
    for t in range(RPT // ZR):
        pltpu.sync_copy(rows_v.at[pl.ds(0, ZR)],
                        agg_sh.at[pl.ds(base + t * ZR, ZR)])
    plsc.subcore_barrier()

    def chunk(j, carry):
        pltpu.sync_copy(xs_hbm.at[src_v.at[j]], rows_v)
        pltpu.sync_copy(rows_v, agg_sh.at[dst_v.at[j]], add=True)
        return carry

    nch = jnp.where(c == 0, NCH0, NCH1)
    lax.fori_loop(0, nch, chunk, None)
    plsc.subcore_barrier()
    for t in range(RPT // ZR):
        sl = pl.ds(base + t * ZR, ZR)
        pltpu.sync_copy(agg_sh.at[sl], rows_v.at[pl.ds(0, ZR)])
        pltpu.sync_copy(rows_v.at[pl.ds(0, ZR)], out_hbm.at[c, sl])


def _make_edge_kernel(NCH0, NCH1, NPAD, D):
    NCHM = max(NCH0, NCH1)
    mesh = plsc.VectorSubcoreMesh(core_axis_name="c", subcore_axis_name="s")
    return pl.kernel(
        functools.partial(_edge_body, NCH0=NCH0, NCH1=NCH1, NPAD=NPAD, D=D),
        out_type=jax.ShapeDtypeStruct((NC, NPAD, D), jnp.float32),
        mesh=mesh,
        compiler_params=pltpu.CompilerParams(needs_layout_passes=False),
        scratch_types=[
            pltpu.VMEM((NCHM, K), jnp.int32),
            pltpu.VMEM((NCHM, K), jnp.int32),
            pltpu.VMEM((K, D), jnp.float32),
            pltpu.VMEM_SHARED((NPAD, D), jnp.float32),
        ],
    )


def _stage1_body(hp_ref, gam_ref, bet_ref, w1_ref, degp_ref, xs_ref, mu_ref,
                 *, N):
    NPAD = hp_ref.shape[0]
    hp = hp_ref[...]
    deg = jnp.sum(degp_ref[...], axis=0)
    norm_out = lax.rsqrt(jnp.maximum(deg[0], 1.0))
    rid = lax.broadcasted_iota(jnp.int32, (NPAD, 1), 0)
    msk = rid < N
    mu = jnp.sum(hp, axis=0, keepdims=True) / N
    xc = jnp.where(msk, hp - mu, 0.0)
    var = jnp.sum(xc * xc, axis=0, keepdims=True) / N
    inv = lax.rsqrt(var + 1e-5)
    hb = xc * inv * gam_ref[...] + bet_ref[...]
    xw = jnp.dot(hb, w1_ref[...], preferred_element_type=jnp.float32)
    xs_ref[...] = xw * norm_out[:, None]
    mu_ref[...] = mu


def _stage2_body(p_ref, hp_ref, gam_ref, bet_ref, w2_ref, b1_ref, degp_ref,
                 h1_ref, xs2_ref, *, N):
    NPAD = hp_ref.shape[0]
    deg = jnp.sum(degp_ref[...], axis=0)
    norm_out = lax.rsqrt(jnp.maximum(deg[0], 1.0))
    norm_in = lax.rsqrt(jnp.maximum(deg[1], 1.0))
    rid = lax.broadcasted_iota(jnp.int32, (NPAD, 1), 0)
    msk = rid < N
    agg = (p_ref[0] + p_ref[1]) * norm_in[:, None] + b1_ref[...]
    h1 = jnp.where(msk, hp_ref[...] + jnp.maximum(agg, 0.0), 0.0)
    mu = jnp.sum(h1, axis=0, keepdims=True) / N
    xc = jnp.where(msk, h1 - mu, 0.0)
    var = jnp.sum(xc * xc, axis=0, keepdims=True) / N
    inv = lax.rsqrt(var + 1e-5)
    hb = xc * inv * gam_ref[...] + bet_ref[...]
    xw = jnp.dot(hb, w2_ref[...], preferred_element_type=jnp.float32)
    h1_ref[...] = h1
    xs2_ref[...] = xw * norm_out[:, None]


def _stage3_body(p_ref, h1_ref, degp_ref, b2_ref, gw_ref, gb_ref, hg_ref,
                 *, N):
    NPAD = h1_ref.shape[0]
    deg = jnp.sum(degp_ref[...], axis=0)
    norm_in = lax.rsqrt(jnp.maximum(deg[1], 1.0))
    rid = lax.broadcasted_iota(jnp.int32, (NPAD, 1), 0)
    msk = rid < N
    agg = (p_ref[0] + p_ref[1]) * norm_in[:, None] + b2_ref[...]
    h2 = jnp.where(msk, h1_ref[...] + jnp.maximum(agg, 0.0), 0.0)
    gate = jnp.sum(h2 * gw_ref[...], axis=1, keepdims=True) + gb_ref[0, 0]
    gate = jnp.where(msk, gate, -1e30)
    m = jnp.max(gate)
    e = jnp.exp(gate - m)
    hg_ref[...] = jnp.sum(e * h2, axis=0, keepdims=True) / jnp.sum(e)


def kernel(h, edge_index, gamma, beta, W1, b1, W2, b2, gate_W, gate_b):
    N, D = h.shape
    E = edge_index.shape[1]
    NW = NC * NS
    NCH = -(-E // (NW * K))
    EP = NW * NCH * K
    NPAD = _round_up(N + 1, NS * 64)

    fill = jnp.full((EP - E,), N, jnp.int32)
    src_rs = jnp.concatenate([edge_index[0], fill]).reshape(NC, NS, NCH, K)
    dst_rs = jnp.concatenate([edge_index[1], fill]).reshape(NC, NS, NCH, K)

    TOT = -(-E // (NS * K))
    NCH0 = int(TOT * 0.49)
    NCH1 = TOT - NCH0
    NCHM = max(NCH0, NCH1)
    cap0 = NS * NCH0 * K
    cap1 = NS * NCH1 * K
    fill_a = jnp.full((cap0 + cap1 - E,), N, jnp.int32)

    def asym(e):
        ep = jnp.concatenate([e, fill_a])
        a0 = ep[:cap0].reshape(NS, NCH0, K)
        a0 = jnp.concatenate(
            [a0, jnp.full((NS, NCHM - NCH0, K), N, jnp.int32)], axis=1)
        a1 = ep[cap0:].reshape(NS, NCH1, K)
        a1 = jnp.concatenate(
            [a1, jnp.full((NS, NCHM - NCH1, K), N, jnp.int32)], axis=1)
        return jnp.stack([a0, a1])

    src_as = asym(edge_index[0])
    dst_as = asym(edge_index[1])

    hp = jnp.zeros((NPAD, D), jnp.float32).at[:N, :].set(h)
    gam = gamma.reshape(1, D)
    bet = beta.reshape(1, D)
    b1r = b1.reshape(1, D)
    b2r = b2.reshape(1, D)
    gw = gate_W.reshape(1, D)
    gb = gate_b.reshape(1, 1)

    degp = _make_deg_kernel(NCH, NPAD)(src_rs, dst_rs)
    degp = degp.reshape(NW, 2, NPAD)

    xs1, mu = pl.pallas_call(
        functools.partial(_stage1_body, N=N),
        out_shape=[jax.ShapeDtypeStruct((NPAD, D), jnp.float32),
                   jax.ShapeDtypeStruct((1, D), jnp.float32)],
    )(hp, gam, bet, W1, degp)

    edge_k = _make_edge_kernel(NCH0, NCH1, NPAD, D)
    p1 = edge_k(xs1, src_as, dst_as)

    h1, xs2 = pl.pallas_call(
        functools.partial(_stage2_body, N=N),
        out_shape=[jax.ShapeDtypeStruct((NPAD, D), jnp.float32),
                   jax.ShapeDtypeStruct((NPAD, D), jnp.float32)],
    )(p1, hp, gam, bet, W2, b1r, degp)

    p2 = edge_k(xs2, src_as, dst_as)

    hg = pl.pallas_call(
        functools.partial(_stage3_body, N=N),
        out_shape=jax.ShapeDtypeStruct((1, D), jnp.float32),
    )(p2, h1, degp, b2r, gw, gb)

    return (hg, mu)


# speedup vs baseline: 1.0480x; 1.0080x over previous
import functools

import jax
import jax.numpy as jnp
from jax import lax
from jax.experimental import pallas as pl
from jax.experimental.pallas import tpu as pltpu
from jax.experimental.pallas import tpu_sc as plsc

NC = 2
NS = 16
L = 16
K = 128


def _round_up(x, m):
    return (x + m - 1) // m * m


_ZCHUNK = 1024


def _deg_body(src_hbm, dst_hbm, out_hbm, idx_v, deg_v, *, NCH, NPAD):
    c = lax.axis_index("c")
    s = lax.axis_index("s")
    ones = jnp.ones((L,), jnp.float32)
    zeros = jnp.zeros((L,), jnp.float32)
    for a in range(2):
        ei = src_hbm if a == 0 else dst_hbm
        pltpu.sync_copy(ei.at[c, s], idx_v)

        def zbody(i, carry):
            for u in range(8):
                deg_v[pl.ds(pl.multiple_of(i * 8 * L + u * L, L), L)] = zeros
            return carry

        lax.fori_loop(0, NPAD // (8 * L), zbody, None)

        def abody(j, carry):
            for k in range(K // L):
                idx = idx_v[j, pl.ds(k * L, L)]
                plsc.addupdate_scatter(deg_v, [idx], ones)
            return carry

        lax.fori_loop(0, NCH, abody, None)
        pltpu.sync_copy(deg_v, out_hbm.at[c, s, a])


def _make_deg_kernel(NCH, NPAD):
    mesh = plsc.VectorSubcoreMesh(core_axis_name="c", subcore_axis_name="s")
    return pl.kernel(
        functools.partial(_deg_body, NCH=NCH, NPAD=NPAD),
        out_type=jax.ShapeDtypeStruct((NC, NS, 2, NPAD), jnp.float32),
        mesh=mesh,
        compiler_params=pltpu.CompilerParams(needs_layout_passes=False),
        scratch_types=[
            pltpu.VMEM((NCH, K), jnp.int32),
            pltpu.VMEM((NPAD,), jnp.float32),
        ],
    )


def _edge_body(xs_hbm, src_hbm, dst_hbm, out_hbm, src_v, dst_v, rows_v,
               agg_sh, *, NCH0, NCH1, NPAD, D):
    RPT = NPAD // NS
    ZR = 64
    c = lax.axis_index("c")
    s = lax.axis_index("s")
    pltpu.sync_copy(src_hbm.at[c, s], src_v)
    pltpu.sync_copy(dst_hbm.at[c, s], dst_v)

    zeros = jnp.zeros((L,), jnp.float32)

    def zbody(r, carry):
        for k in range(D // L):
            rows_v[r, pl.ds(k * L, L)] = zeros
        return carry

    lax.fori_loop(0, ZR, zbody, None)
    base = s * RPT
    for t in range(RPT // ZR):
        pltpu.sync_copy(rows_v.at[pl.ds(0, ZR)],
                        agg_sh.at[pl.ds(base + t * ZR, ZR)])
    plsc.subcore_barrier()

    def chunk(j, carry):
        pltpu.sync_copy(xs_hbm.at[src_v.at[j]], rows_v)
        pltpu.sync_copy(rows_v, agg_sh.at[dst_v.at[j]], add=True)
        return carry

    nch = jnp.where(c == 0, NCH0, NCH1)
    lax.fori_loop(0, nch, chunk, None)
    plsc.subcore_barrier()
    for t in range(RPT // K):
        sl = pl.ds(base + t * K, K)
        pltpu.sync_copy(agg_sh.at[sl], out_hbm.at[c, sl])


def _make_edge_kernel(NCH0, NCH1, NPAD, D):
    NCHM = max(NCH0, NCH1)
    mesh = plsc.VectorSubcoreMesh(core_axis_name="c", subcore_axis_name="s")
    return pl.kernel(
        functools.partial(_edge_body, NCH0=NCH0, NCH1=NCH1, NPAD=NPAD, D=D),
        out_type=jax.ShapeDtypeStruct((NC, NPAD, D), jnp.float32),
        mesh=mesh,
        compiler_params=pltpu.CompilerParams(needs_layout_passes=False),
        scratch_types=[
            pltpu.VMEM((NCHM, K), jnp.int32),
            pltpu.VMEM((NCHM, K), jnp.int32),
            pltpu.VMEM((K, D), jnp.float32),
            pltpu.VMEM_SHARED((NPAD, D), jnp.float32),
        ],
    )


def _stage1_body(hp_ref, gam_ref, bet_ref, w1_ref, degp_ref, xs_ref, mu_ref,
                 *, N):
    NPAD = hp_ref.shape[0]
    hp = hp_ref[...]
    deg = jnp.sum(degp_ref[...], axis=0)
    norm_out = lax.rsqrt(jnp.maximum(deg[0], 1.0))
    rid = lax.broadcasted_iota(jnp.int32, (NPAD, 1), 0)
    msk = rid < N
    mu = jnp.sum(hp, axis=0, keepdims=True) / N
    xc = jnp.where(msk, hp - mu, 0.0)
    var = jnp.sum(xc * xc, axis=0, keepdims=True) / N
    inv = lax.rsqrt(var + 1e-5)
    hb = xc * inv * gam_ref[...] + bet_ref[...]
    xw = jnp.dot(hb, w1_ref[...], preferred_element_type=jnp.float32)
    xs_ref[...] = xw * norm_out[:, None]
    mu_ref[...] = mu


def _stage2_body(p_ref, hp_ref, gam_ref, bet_ref, w2_ref, b1_ref, degp_ref,
                 h1_ref, xs2_ref, *, N):
    NPAD = hp_ref.shape[0]
    deg = jnp.sum(degp_ref[...], axis=0)
    norm_out = lax.rsqrt(jnp.maximum(deg[0], 1.0))
    norm_in = lax.rsqrt(jnp.maximum(deg[1], 1.0))
    rid = lax.broadcasted_iota(jnp.int32, (NPAD, 1), 0)
    msk = rid < N
    agg = (p_ref[0] + p_ref[1]) * norm_in[:, None] + b1_ref[...]
    h1 = jnp.where(msk, hp_ref[...] + jnp.maximum(agg, 0.0), 0.0)
    mu = jnp.sum(h1, axis=0, keepdims=True) / N
    xc = jnp.where(msk, h1 - mu, 0.0)
    var = jnp.sum(xc * xc, axis=0, keepdims=True) / N
    inv = lax.rsqrt(var + 1e-5)
    hb = xc * inv * gam_ref[...] + bet_ref[...]
    xw = jnp.dot(hb, w2_ref[...], preferred_element_type=jnp.float32)
    h1_ref[...] = h1
    xs2_ref[...] = xw * norm_out[:, None]


def _stage3_body(p_ref, h1_ref, degp_ref, b2_ref, gw_ref, gb_ref, hg_ref,
                 *, N):
    NPAD = h1_ref.shape[0]
    deg = jnp.sum(degp_ref[...], axis=0)
    norm_in = lax.rsqrt(jnp.maximum(deg[1], 1.0))
    rid = lax.broadcasted_iota(jnp.int32, (NPAD, 1), 0)
    msk = rid < N
    agg = (p_ref[0] + p_ref[1]) * norm_in[:, None] + b2_ref[...]
    h2 = jnp.where(msk, h1_ref[...] + jnp.maximum(agg, 0.0), 0.0)
    gate = jnp.sum(h2 * gw_ref[...], axis=1, keepdims=True) + gb_ref[0, 0]
    gate = jnp.where(msk, gate, -1e30)
    m = jnp.max(gate)
    e = jnp.exp(gate - m)
    hg_ref[...] = jnp.sum(e * h2, axis=0, keepdims=True) / jnp.sum(e)


def kernel(h, edge_index, gamma, beta, W1, b1, W2, b2, gate_W, gate_b):
    N, D = h.shape
    E = edge_index.shape[1]
    NW = NC * NS
    NCH = -(-E // (NW * K))
    EP = NW * NCH * K
    NPAD = _round_up(N + 1, NS * 64)

    fill = jnp.full((EP - E,), N, jnp.int32)
    src_rs = jnp.concatenate([edge_index[0], fill]).reshape(NC, NS, NCH, K)
    dst_rs = jnp.concatenate([edge_index[1], fill]).reshape(NC, NS, NCH, K)

    TOT = -(-E // (NS * K))
    NCH0 = int(TOT * 0.49)
    NCH1 = TOT - NCH0
    NCHM = max(NCH0, NCH1)
    cap0 = NS * NCH0 * K
    cap1 = NS * NCH1 * K
    fill_a = jnp.full((cap0 + cap1 - E,), N, jnp.int32)

    def asym(e):
        ep = jnp.concatenate([e, fill_a])
        a0 = ep[:cap0].reshape(NS, NCH0, K)
        a0 = jnp.concatenate(
            [a0, jnp.full((NS, NCHM - NCH0, K), N, jnp.int32)], axis=1)
        a1 = ep[cap0:].reshape(NS, NCH1, K)
        a1 = jnp.concatenate(
            [a1, jnp.full((NS, NCHM - NCH1, K), N, jnp.int32)], axis=1)
        return jnp.stack([a0, a1])

    src_as = asym(edge_index[0])
    dst_as = asym(edge_index[1])

    hp = jnp.zeros((NPAD, D), jnp.float32).at[:N, :].set(h)
    gam = gamma.reshape(1, D)
    bet = beta.reshape(1, D)
    b1r = b1.reshape(1, D)
    b2r = b2.reshape(1, D)
    gw = gate_W.reshape(1, D)
    gb = gate_b.reshape(1, 1)

    degp = _make_deg_kernel(NCH, NPAD)(src_rs, dst_rs)
    degp = degp.reshape(NW, 2, NPAD)

    xs1, mu = pl.pallas_call(
        functools.partial(_stage1_body, N=N),
        out_shape=[jax.ShapeDtypeStruct((NPAD, D), jnp.float32),
                   jax.ShapeDtypeStruct((1, D), jnp.float32)],
    )(hp, gam, bet, W1, degp)

    edge_k = _make_edge_kernel(NCH0, NCH1, NPAD, D)
    p1 = edge_k(xs1, src_as, dst_as)

    h1, xs2 = pl.pallas_call(
        functools.partial(_stage2_body, N=N),
        out_shape=[jax.ShapeDtypeStruct((NPAD, D), jnp.float32),
                   jax.ShapeDtypeStruct((NPAD, D), jnp.float32)],
    )(p1, hp, gam, bet, W2, b1r, degp)

    p2 = edge_k(xs2, src_as, dst_as)

    hg = pl.pallas_call(
        functools.partial(_stage3_body, N=N),
        out_shape=jax.ShapeDtypeStruct((1, D), jnp.float32),
    )(p2, h1, degp, b2r, gw, gb)

    return (hg, mu)
